# baseline (speedup 1.0000x reference)
import functools

import jax
import jax.numpy as jnp
from jax import lax
from jax.experimental import pallas as pl
from jax.experimental.pallas import tpu as pltpu
from jax.experimental.pallas import tpu_sc as plsc

SEQ = 16384
TOK = 50
EMBED = 64
NTOTAL = SEQ * TOK

_info = plsc.get_sparse_core_info()
NC = _info.num_cores
NS = _info.num_subcores
NW = NC * NS
BPW = NTOTAL // NW
CHUNK = 128
NCHUNK = BPW // CHUNK
NBUF = 5
HCHUNK = CHUNK // 2
LANES = 16
MOVES = CHUNK * EMBED // LANES
MPG = 32

_mesh = plsc.VectorSubcoreMesh(core_axis_name="c", subcore_axis_name="s")


@functools.partial(
    pl.kernel,
    mesh=_mesh,
    out_type=jax.ShapeDtypeStruct((NTOTAL // 2, 2 * EMBED), jnp.float32),
    scratch_types=[
        pltpu.VMEM((NCHUNK, CHUNK), jnp.int32),
        pltpu.VMEM((NBUF, CHUNK, EMBED), jnp.float32),
        pltpu.VMEM((NBUF, HCHUNK, 2 * EMBED), jnp.float32),
        pltpu.SemaphoreType.DMA((NBUF,)),
        pltpu.SemaphoreType.DMA((NBUF,)),
    ],
    compiler_params=pltpu.CompilerParams(use_tc_tiling_on_sc=False),
)
def _gather_kernel(idx_hbm, table_hbm, out_hbm, idx_v, rows_v, packed_v,
                   gsem, ssem):
    wid = lax.axis_index("s") * NC + lax.axis_index("c")
    base2 = wid * (BPW // 2)

    pltpu.sync_copy(idx_hbm.at[wid], idx_v)

    def gather_copy(s, b):
        return pltpu.make_async_copy(
            table_hbm.at[idx_v.at[s]], rows_v.at[b], gsem.at[b])

    def store_copy(s, b):
        return pltpu.make_async_copy(
            packed_v.at[b],
            out_hbm.at[pl.ds(base2 + s * HCHUNK, HCHUNK)],
            ssem.at[b])

    def repack(b):
        src = rows_v.at[b]
        dst = packed_v.at[b]

        def rep_body(kk, carry):
            k0 = kk * MPG
            for j in range(MPG):
                sr = k0 // 4 + j // 4
                sc = (j % 4) * LANES
                dr = k0 // 8 + j // 8
                dc = (j % 8) * LANES
                dst[dr, pl.ds(dc, LANES)] = src[sr, pl.ds(sc, LANES)]
            return carry

        lax.fori_loop(0, MOVES // MPG, rep_body, 0)

    def step_a(s, b):
        gather_copy(s, b).wait()
        repack(b)
        store_copy(s, b).start()

    def step_b(s, b, guard):
        store_copy(s, b).wait()
        if guard:

            @pl.when(s + NBUF < NCHUNK)
            def _():
                gather_copy(s + NBUF, b).start()

        else:
            gather_copy(s + NBUF, b).start()

    for b in range(NBUF):
        gather_copy(b, b).start()

    step_a(0, 0)
    for b in range(1, NBUF):
        step_b(b - 1, b - 1, guard=False)
        step_a(b, b)

    def outer(o, carry):
        s0 = o * NBUF
        for b in range(NBUF):
            s = s0 + b
            bp = (b - 1) % NBUF
            step_b(s - 1, bp, guard=True)
            step_a(s, b)
        return carry

    lax.fori_loop(1, NCHUNK // NBUF, outer, 0)

    store_copy(NCHUNK - 1, (NCHUNK - 1) % NBUF).wait()


def kernel(x, table):
    idx = x.reshape(NW, NCHUNK, CHUNK).astype(jnp.int32)
    out = _gather_kernel(idx, table)
    return out.reshape(SEQ, TOK, EMBED)


# speedup vs baseline: 1.2511x; 1.2511x over previous
import functools

import jax
import jax.numpy as jnp
from jax import lax
from jax.experimental import pallas as pl
from jax.experimental.pallas import tpu as pltpu
from jax.experimental.pallas import tpu_sc as plsc
from jax.experimental.layout import Format, Layout, with_layout_constraint

SEQ = 16384
TOK = 50
EMBED = 64
NTOTAL = SEQ * TOK

_info = plsc.get_sparse_core_info()
NC = _info.num_cores
NS = _info.num_subcores
NW = NC * NS
BPW = NTOTAL // NW
CHUNK = 128
NCHUNK = BPW // CHUNK
NBUF = 5
HCHUNK = CHUNK // 2
LANES = 16
MOVES = CHUNK * EMBED // LANES
MPG = 32

_mesh = plsc.VectorSubcoreMesh(core_axis_name="c", subcore_axis_name="s")


@functools.partial(
    pl.kernel,
    mesh=_mesh,
    out_type=jax.ShapeDtypeStruct((NTOTAL // 2, 2 * EMBED), jnp.float32),
    scratch_types=[
        pltpu.VMEM((NCHUNK, CHUNK), jnp.int32),
        pltpu.VMEM((NBUF, CHUNK, EMBED), jnp.float32),
        pltpu.VMEM((NBUF, HCHUNK, 2 * EMBED), jnp.float32),
        pltpu.SemaphoreType.DMA((NBUF,)),
        pltpu.SemaphoreType.DMA((NBUF,)),
    ],
    compiler_params=pltpu.CompilerParams(use_tc_tiling_on_sc=False),
)
def _gather_kernel(idx_hbm, table_hbm, out_hbm, idx_v, rows_v, packed_v,
                   gsem, ssem):
    wid = lax.axis_index("s") * NC + lax.axis_index("c")
    base2 = wid * (BPW // 2)

    pltpu.sync_copy(idx_hbm.at[wid], idx_v)

    def gather_copy(s, b):
        return pltpu.make_async_copy(
            table_hbm.at[idx_v.at[s]], rows_v.at[b], gsem.at[b])

    def store_copy(s, b):
        return pltpu.make_async_copy(
            packed_v.at[b],
            out_hbm.at[pl.ds(base2 + s * HCHUNK, HCHUNK)],
            ssem.at[b])

    def repack(b):
        src = rows_v.at[b]
        dst = packed_v.at[b]

        def rep_body(kk, carry):
            k0 = kk * MPG
            for j in range(MPG):
                sr = k0 // 4 + j // 4
                sc = (j % 4) * LANES
                dr = k0 // 8 + j // 8
                dc = (j % 8) * LANES
                dst[dr, pl.ds(dc, LANES)] = src[sr, pl.ds(sc, LANES)]
            return carry

        lax.fori_loop(0, MOVES // MPG, rep_body, 0)

    def step_a(s, b):
        gather_copy(s, b).wait()
        repack(b)
        store_copy(s, b).start()

    def step_b(s, b, guard):
        store_copy(s, b).wait()
        if guard:

            @pl.when(s + NBUF < NCHUNK)
            def _():
                gather_copy(s + NBUF, b).start()

        else:
            gather_copy(s + NBUF, b).start()

    for b in range(NBUF):
        gather_copy(b, b).start()

    step_a(0, 0)
    for b in range(1, NBUF):
        step_b(b - 1, b - 1, guard=False)
        step_a(b, b)

    def outer(o, carry):
        s0 = o * NBUF
        for b in range(NBUF):
            s = s0 + b
            bp = (b - 1) % NBUF
            step_b(s - 1, bp, guard=True)
            step_a(s, b)
        return carry

    lax.fori_loop(1, NCHUNK // NBUF, outer, 0)

    store_copy(NCHUNK - 1, (NCHUNK - 1) % NBUF).wait()


def kernel(x, table):
    idx = x.reshape(NW, NCHUNK, CHUNK).astype(jnp.int32)
    table_sc = with_layout_constraint(
        table, Layout(major_to_minor=(0, 1), tiling=((16,),)))
    out = _gather_kernel(idx, table_sc)
    return out.reshape(SEQ, TOK, EMBED)
